# baseline (speedup 1.0000x reference)
import functools

import jax
import jax.numpy as jnp
from jax import lax
from jax.experimental import pallas as pl
from jax.experimental.pallas import tpu as pltpu
from jax.experimental.pallas import tpu_sc as plsc

_NUM_CORES = 2
_NUM_SUBCORES = 16
_NW = _NUM_CORES * _NUM_SUBCORES
_LANES = 16


@functools.lru_cache(maxsize=None)
def _build(batch, seq_len, dim):
  s_per_w = seq_len // _NW
  n_chunks = batch
  rows_per_w = batch * s_per_w
  vecs_per_row = dim // _LANES

  mesh = plsc.VectorSubcoreMesh(core_axis_name="c", subcore_axis_name="s")

  @functools.partial(
      pl.kernel,
      mesh=mesh,
      out_type=jax.ShapeDtypeStruct((batch, seq_len, dim), jnp.float32),
      scratch_types=(
          [pltpu.VMEM((n_chunks, s_per_w), jnp.int32),
           pltpu.VMEM((rows_per_w, dim), jnp.float32),
           pltpu.VMEM((s_per_w, dim), jnp.float32)]
          + [pltpu.SemaphoreType.DMA] * (2 * n_chunks + 3)
      ),
  )
  def tok_pos_embed(idx_hbm, tok_hbm, pos_hbm, out_hbm,
                    idx_v, rows_v, pos_v, *sems):
    psem0 = sems[0]
    psem1 = sems[1]
    osem = sems[2]
    isems = sems[3:3 + n_chunks]
    gsems = sems[3 + n_chunks:3 + 2 * n_chunks]

    wid = lax.axis_index("s") * _NUM_CORES + lax.axis_index("c")
    s_base = wid * s_per_w

    idx_cps = [
        pltpu.async_copy(idx_hbm.at[j, pl.ds(s_base, s_per_w)],
                         idx_v.at[j], isems[j])
        for j in range(n_chunks)
    ]
    pos_cp0 = pltpu.async_copy(pos_hbm.at[pl.ds(s_base, s_per_w)],
                               rows_v.at[pl.ds(0, s_per_w)], psem0)
    pos_cp1 = pltpu.async_copy(pos_hbm.at[pl.ds(s_base, s_per_w)],
                               pos_v, psem1)

    def fill_chunk(j):
      def rep_row(r, carry):
        for c in range(vecs_per_row):
          sl = pl.ds(c * _LANES, _LANES)
          rows_v[j * s_per_w + r, sl] = pos_v[r, sl]
        return carry
      lax.fori_loop(0, s_per_w, rep_row, 0)

    def gather_chunk(j):
      return pltpu.async_copy(tok_hbm.at[idx_v.at[j]],
                              rows_v.at[pl.ds(j * s_per_w, s_per_w)],
                              gsems[j], add=True)

    def store_chunk(j):
      return pltpu.async_copy(rows_v.at[pl.ds(j * s_per_w, s_per_w)],
                              out_hbm.at[j, pl.ds(s_base, s_per_w)],
                              osem)

    pos_cp0.wait()
    idx_cps[0].wait()
    gathers = [gather_chunk(0)]
    stores = []
    pos_cp1.wait()
    for j in range(1, n_chunks):
      fill_chunk(j)
      idx_cps[j].wait()
      gathers.append(gather_chunk(j))
      gathers[j - 1].wait()
      stores.append(store_chunk(j - 1))
    gathers[n_chunks - 1].wait()
    stores.append(store_chunk(n_chunks - 1))
    for s in stores:
      s.wait()

  return tok_pos_embed


def kernel(x, tok_table, pos_table):
  batch, seq_len = x.shape
  _, dim = tok_table.shape
  fn = _build(batch, seq_len, dim)
  return fn(x, tok_table, pos_table)


# speedup vs baseline: 1.0037x; 1.0037x over previous
import functools

import jax
import jax.numpy as jnp
from jax import lax
from jax.experimental import pallas as pl
from jax.experimental.pallas import tpu as pltpu
from jax.experimental.pallas import tpu_sc as plsc

_NUM_CORES = 2
_NUM_SUBCORES = 16
_NW = _NUM_CORES * _NUM_SUBCORES
_LANES = 16


@functools.lru_cache(maxsize=None)
def _build(batch, seq_len, dim):
  s_per_w = seq_len // _NW
  n_chunks = batch
  rows_per_w = batch * s_per_w
  vecs_per_row = dim // _LANES

  mesh = plsc.VectorSubcoreMesh(core_axis_name="c", subcore_axis_name="s")

  @functools.partial(
      pl.kernel,
      mesh=mesh,
      out_type=jax.ShapeDtypeStruct((batch, seq_len, dim), jnp.float32),
      scratch_types=(
          [pltpu.VMEM((n_chunks, s_per_w), jnp.int32),
           pltpu.VMEM((rows_per_w, dim), jnp.float32),
           pltpu.VMEM((s_per_w, dim), jnp.float32)]
          + [pltpu.SemaphoreType.DMA] * (3 * n_chunks + 3)
      ),
  )
  def tok_pos_embed(idx_hbm, tok_hbm, pos_hbm, out_hbm,
                    idx_v, rows_v, pos_v, *sems):
    psem0 = sems[0]
    psem1 = sems[1]
    osem = sems[2]
    isems = sems[3:3 + n_chunks]
    gsems = sems[3 + n_chunks:3 + 3 * n_chunks]

    wid = lax.axis_index("s") * _NUM_CORES + lax.axis_index("c")
    s_base = wid * s_per_w

    idx_cps = [
        pltpu.async_copy(idx_hbm.at[j, pl.ds(s_base, s_per_w)],
                         idx_v.at[j], isems[j])
        for j in range(n_chunks)
    ]
    pos_cp0 = pltpu.async_copy(pos_hbm.at[pl.ds(s_base, s_per_w)],
                               rows_v.at[pl.ds(0, s_per_w)], psem0)
    pos_cp1 = pltpu.async_copy(pos_hbm.at[pl.ds(s_base, s_per_w)],
                               pos_v, psem1)

    def fill_chunk(j):
      def rep_row(r, carry):
        for c in range(vecs_per_row):
          sl = pl.ds(c * _LANES, _LANES)
          rows_v[j * s_per_w + r, sl] = pos_v[r, sl]
        return carry
      lax.fori_loop(0, s_per_w, rep_row, 0)

    half = s_per_w // 2
    gathers = []
    stores = []

    def gather_half(j, h):
      return pltpu.async_copy(
          tok_hbm.at[idx_v.at[j, pl.ds(h * half, half)]],
          rows_v.at[pl.ds(j * s_per_w + h * half, half)],
          gsems[2 * j + h], add=True)

    def store_half(j, h):
      return pltpu.async_copy(
          rows_v.at[pl.ds(j * s_per_w + h * half, half)],
          out_hbm.at[j, pl.ds(s_base + h * half, half)],
          osem)

    pos_cp0.wait()
    idx_cps[0].wait()
    gathers += [gather_half(0, 0), gather_half(0, 1)]
    pos_cp1.wait()
    for j in range(1, n_chunks):
      fill_chunk(j)
      idx_cps[j].wait()
      gathers += [gather_half(j, 0), gather_half(j, 1)]
      gathers[2 * (j - 1)].wait()
      stores.append(store_half(j - 1, 0))
      gathers[2 * (j - 1) + 1].wait()
      stores.append(store_half(j - 1, 1))
    gathers[2 * (n_chunks - 1)].wait()
    stores.append(store_half(n_chunks - 1, 0))
    gathers[2 * (n_chunks - 1) + 1].wait()
    stores.append(store_half(n_chunks - 1, 1))
    for s in stores:
      s.wait()

  return tok_pos_embed


def kernel(x, tok_table, pos_table):
  batch, seq_len = x.shape
  _, dim = tok_table.shape
  fn = _build(batch, seq_len, dim)
  return fn(x, tok_table, pos_table)
